# baseline (speedup 1.0000x reference)
import jax
import jax.numpy as jnp
from jax import lax
from jax.experimental import pallas as pl
from jax.experimental.pallas import tpu as pltpu
from jax.experimental.pallas import tpu_sc as plsc

N = 10000
E = 320000
D = 128
H = 128
DE = 16
CP = 16

NC, NS = 2, 16
NW = NC * NS
NH = 2
EH = E // NH
CH = 128
NCHT = EH // CH
KMAX = -(-NCHT // NW)
BE = 1280
GB = BE // CH
NBLK = EH // BE
L = 16

_f32 = jnp.float32
_i32 = jnp.int32


def _worker_id():
    return lax.axis_index("s") * NC + lax.axis_index("c")


def _mesh():
    return plsc.VectorSubcoreMesh(core_axis_name="c", subcore_axis_name="s",
                                  num_cores=NC, num_subcores=NS)


def _make_gather_body(eoff):
    def body(nodes_h, cx_h, cy_h, cz_h, row_h, col_h,
             src_o, dst_o, rad_o,
             cx, cy, cz, ir, ic, b1, b2, rbuf, s1, s2):
        wid = _worker_id()
        s = lax.axis_index("s")
        pltpu.sync_copy(cx_h, cx)
        pltpu.sync_copy(cy_h, cy)
        pltpu.sync_copy(cz_h, cz)

        def loop(k, _):
            cid = k * NW + wid

            @pl.when(cid < NCHT)
            def _():
                base = cid * CH
                pltpu.sync_copy(row_h.at[pl.ds(eoff + base, CH)], ir)
                pltpu.sync_copy(col_h.at[pl.ds(eoff + base, CH)], ic)
                d1 = pltpu.async_copy(nodes_h.at[ir], b1, s1)
                d2 = pltpu.async_copy(nodes_h.at[ic], b2, s2)
                for g in range(CH // L):
                    ivr = ir[pl.ds(g * L, L)]
                    ivc = ic[pl.ds(g * L, L)]
                    dx = plsc.load_gather(cx, [ivr]) - plsc.load_gather(cx, [ivc])
                    dy = plsc.load_gather(cy, [ivr]) - plsc.load_gather(cy, [ivc])
                    dz = plsc.load_gather(cz, [ivr]) - plsc.load_gather(cz, [ivc])
                    rbuf[0, pl.ds(g * L, L)] = dx * dx + dy * dy + dz * dz
                pltpu.sync_copy(rbuf, rad_o.at[cid // GB].at[pl.ds(cid % GB, 1)])
                d1.wait()
                d2.wait()
                pltpu.sync_copy(b1, src_o.at[pl.ds(base, CH)])
                pltpu.sync_copy(b2, dst_o.at[pl.ds(base, CH)])
            return _

        lax.fori_loop(0, KMAX, loop, None)

    return body


def _sc_gather(nodes, cx, cy, cz, row, col, eoff):
    fn = pl.kernel(
        _make_gather_body(eoff),
        out_type=[
            jax.ShapeDtypeStruct((EH, D), _f32),
            jax.ShapeDtypeStruct((EH, D), _f32),
            jax.ShapeDtypeStruct((NBLK, GB, CH), _f32),
        ],
        mesh=_mesh(),
        scratch_types=[
            pltpu.VMEM((N,), _f32), pltpu.VMEM((N,), _f32),
            pltpu.VMEM((N,), _f32),
            pltpu.VMEM((CH,), _i32), pltpu.VMEM((CH,), _i32),
            pltpu.VMEM((CH, D), _f32), pltpu.VMEM((CH, D), _f32),
            pltpu.VMEM((1, CH), _f32),
            pltpu.SemaphoreType.DMA, pltpu.SemaphoreType.DMA,
        ],
        compiler_params=pltpu.CompilerParams(needs_layout_passes=False),
    )
    return fn(nodes, cx, cy, cz, row, col)


SUB = 2
BS = BE // SUB
GS = GB // SUB


def _edge_mlp_body(src_r, dst_r, rad_r, ea_r, Z_r, oh_r, Zt_r,
                   Wsa_r, Wda_r, Wea_r, b1a_r, rw_r, eW2_r, eb2_r,
                   aW2_r, ab2_r, cW1_r, cb1_r, cW2_r, cb2_r,
                   ef_o, c8_o):
    bf = jnp.bfloat16
    Z = Z_r[...]
    oh = oh_r[...]
    Zt = Zt_r[...]

    def mm(a, b):
        return jnp.dot(a.astype(bf), b, preferred_element_type=_f32)

    def tshrink(x):
        return x - jnp.tanh(x)

    for sub in range(SUB):
        rs = pl.ds(sub * BS, BS)
        src = src_r[rs, :].astype(bf)
        dst = dst_r[rs, :].astype(bf)
        ea = ea_r[rs, :].astype(bf)
        rad8 = rad_r[0, pl.ds(sub * GS, GS), :]

        M = mm(Z, rad8.astype(bf))
        radcol = jnp.sum(M * oh, axis=1, keepdims=True)

        ha = (mm(src, Wsa_r[...]) + mm(dst, Wda_r[...]) + mm(ea, Wea_r[...])
              + radcol * rw_r[...] + b1a_r[...])
        ha = tshrink(ha)
        h = ha[:, :H]
        a = ha[:, H:]
        h = tshrink(mm(h, eW2_r[...]) + eb2_r[...])
        a = mm(a, aW2_r[...]) + ab2_r[...]
        a = 1.0 / (1.0 + jnp.exp(-a))

        ef = h * a
        cc = tshrink(mm(ef, cW1_r[...]) + cb1_r[...])
        cc = mm(cc, cW2_r[...]) + cb2_r[...]

        ef_o[rs, :] = ef
        CC = (cc * oh).astype(bf)
        c8_o[0, pl.ds(sub * GS, GS), :] = mm(Zt, CC)


def _edge_mlp(src, dst, rad8, edge_attr, w, hoff):
    grid = (NBLK,)

    def eb(i):
        return (i, 0)

    def wspec(shape):
        return pl.BlockSpec(shape, lambda i: (0, 0))

    in_specs = [pl.BlockSpec((BE, D), eb), pl.BlockSpec((BE, D), eb),
                pl.BlockSpec((1, GB, CH), lambda i: (i, 0, 0)),
                pl.BlockSpec((BE, DE), lambda i: (i + hoff, 0)),
                wspec((BS, GS)), wspec((BS, CH)), wspec((GS, BS)),
                wspec((D, 2 * H)), wspec((D, 2 * H)), wspec((DE, 2 * H)),
                wspec((1, 2 * H)), wspec((1, 2 * H)),
                wspec((H, H)), wspec((1, H)),
                wspec((H, 1)), wspec((1, 1)),
                wspec((H, H)), wspec((1, H)), wspec((H, 1)), wspec((1, 1))]
    out_specs = [pl.BlockSpec((BE, D), eb),
                 pl.BlockSpec((1, GB, CH), lambda i: (i, 0, 0))]
    return pl.pallas_call(
        _edge_mlp_body,
        grid=grid,
        in_specs=in_specs,
        out_specs=out_specs,
        out_shape=[jax.ShapeDtypeStruct((EH, D), _f32),
                   jax.ShapeDtypeStruct((NBLK, GB, CH), _f32)],
        compiler_params=pltpu.CompilerParams(
            dimension_semantics=("arbitrary",)),
    )(src, dst, rad8, edge_attr, *w)


def _pack_consts():
    bi = jnp.arange(BS) // CH
    Z = (bi[:, None] == jnp.arange(GS)[None, :]).astype(jnp.bfloat16)
    oh = (jnp.arange(CH)[None, :] == (jnp.arange(BS) % CH)[:, None]).astype(_f32)
    Zt = (jnp.arange(GS)[:, None] == bi[None, :]).astype(jnp.bfloat16)
    return Z, oh, Zt


RP = 624
RREM = N - NS * RP


def _acc_init(z_h, acc, s):
    pltpu.sync_copy(z_h.at[pl.ds(s * RP, RP)], acc.at[pl.ds(s * RP, RP)])

    @pl.when(s == NS - 1)
    def _():
        pltpu.sync_copy(z_h.at[pl.ds(NS * RP, RREM)],
                        acc.at[pl.ds(NS * RP, RREM)])


def _acc_writeout(acc, out_o, c, s):
    pltpu.sync_copy(acc.at[pl.ds(s * RP, RP)],
                    out_o.at[c].at[pl.ds(s * RP, RP)])

    @pl.when(s == NS - 1)
    def _():
        pltpu.sync_copy(acc.at[pl.ds(NS * RP, RREM)],
                        out_o.at[c].at[pl.ds(NS * RP, RREM)])


def _make_scatter_f_body(eoff):
    def body(v_h, row_h, z_h, out_o, acc, idx, buf, s1):
        c = lax.axis_index("c")
        s = lax.axis_index("s")
        wid = _worker_id()

        _acc_init(z_h, acc, s)
        plsc.subcore_barrier()

        def loop(k, _):
            cid = k * NW + wid

            @pl.when(cid < NCHT)
            def _():
                base = cid * CH
                pltpu.sync_copy(row_h.at[pl.ds(eoff + base, CH)], idx)
                pltpu.async_copy(v_h.at[pl.ds(base, CH)], buf, s1).wait()
                pltpu.sync_copy(buf, acc.at[idx], add=True)
            return _

        lax.fori_loop(0, KMAX, loop, None)
        plsc.subcore_barrier()
        _acc_writeout(acc, out_o, c, s)

    return body


def _sc_scatter_f(ef, row, zf, eoff):
    fn = pl.kernel(
        _make_scatter_f_body(eoff),
        out_type=[jax.ShapeDtypeStruct((NC, N, D), _f32)],
        mesh=_mesh(),
        scratch_types=[
            pltpu.VMEM_SHARED((N, D), _f32),
            pltpu.VMEM((CH,), _i32),
            pltpu.VMEM((CH, D), _f32),
            pltpu.SemaphoreType.DMA,
        ],
    )
    return fn(ef, row, zf)[0]


def _make_scatter_c_body(eoff):
    def body(c8_h, row_h, col_h, cx_h, cy_h, cz_h, z_h, out_o,
             acc, cx, cy, cz, ir, ic, cbuf, trbuf, s1):
        c = lax.axis_index("c")
        s = lax.axis_index("s")
        wid = _worker_id()

        pltpu.sync_copy(cx_h, cx)
        pltpu.sync_copy(cy_h, cy)
        pltpu.sync_copy(cz_h, cz)
        _acc_init(z_h, acc, s)
        plsc.subcore_barrier()

        iota = lax.iota(_i32, L)

        def loop(k, _):
            cid = k * NW + wid

            @pl.when(cid < NCHT)
            def _():
                base = cid * CH
                pltpu.sync_copy(row_h.at[pl.ds(eoff + base, CH)], ir)
                pltpu.sync_copy(col_h.at[pl.ds(eoff + base, CH)], ic)
                pltpu.sync_copy(c8_h.at[cid // GB].at[pl.ds(cid % GB, 1)], cbuf)
                for g in range(CH // L):
                    ivr = ir[pl.ds(g * L, L)]
                    ivc = ic[pl.ds(g * L, L)]
                    cv = cbuf[0, pl.ds(g * L, L)]
                    tx = (plsc.load_gather(cx, [ivr])
                          - plsc.load_gather(cx, [ivc])) * cv
                    ty = (plsc.load_gather(cy, [ivr])
                          - plsc.load_gather(cy, [ivc])) * cv
                    tz = (plsc.load_gather(cz, [ivr])
                          - plsc.load_gather(cz, [ivc])) * cv
                    ridx = iota + (g * L)
                    plsc.store_scatter(trbuf, [ridx, jnp.full((L,), 0, _i32)], tx)
                    plsc.store_scatter(trbuf, [ridx, jnp.full((L,), 1, _i32)], ty)
                    plsc.store_scatter(trbuf, [ridx, jnp.full((L,), 2, _i32)], tz)
                pltpu.sync_copy(trbuf, acc.at[ir], add=True)
            return _

        lax.fori_loop(0, KMAX, loop, None)
        plsc.subcore_barrier()
        _acc_writeout(acc, out_o, c, s)

    return body


def _sc_scatter_c(c8, row, col, cx, cy, cz, zc, eoff):
    fn = pl.kernel(
        _make_scatter_c_body(eoff),
        out_type=[jax.ShapeDtypeStruct((NC, N, CP), _f32)],
        mesh=_mesh(),
        scratch_types=[
            pltpu.VMEM_SHARED((N, CP), _f32),
            pltpu.VMEM((N,), _f32), pltpu.VMEM((N,), _f32),
            pltpu.VMEM((N,), _f32),
            pltpu.VMEM((CH,), _i32), pltpu.VMEM((CH,), _i32),
            pltpu.VMEM((1, CH), _f32),
            pltpu.VMEM((CH, CP), _f32),
            pltpu.SemaphoreType.DMA,
        ],
        compiler_params=pltpu.CompilerParams(use_tc_tiling_on_sc=False,
                                             needs_layout_passes=False),
    )
    return fn(c8, row, col, cx, cy, cz, zc)[0]


def _node_body(nodes_r, coordp_r, f0_r, f1_r, c0_r, c1_r,
               nW1a_r, nW1b_r, nb1_r, nW2_r, nb2_r,
               nodes_o, coordp_o):
    nodes = nodes_r[...]
    aggf = f0_r[0] + f0_r[1] + f1_r[0] + f1_r[1]

    def mm(a, b):
        return jnp.dot(a.astype(jnp.bfloat16), b, preferred_element_type=_f32)

    n = mm(nodes, nW1a_r[...]) + mm(aggf, nW1b_r[...]) + nb1_r[...]
    n = n - jnp.tanh(n)
    n = mm(n, nW2_r[...]) + nb2_r[...]
    nodes_o[...] = nodes + n
    coordp_o[...] = coordp_r[...] + c0_r[0] + c0_r[1] + c1_r[0] + c1_r[1]


def _node_mlp(nodes, coordp, aggf, aggc, nW1a, nW1b, nb1, nW2, nb2):
    BN = 2000
    grid = (N // BN,)
    return pl.pallas_call(
        _node_body,
        grid=grid,
        in_specs=[
            pl.BlockSpec((BN, D), lambda i: (i, 0)),
            pl.BlockSpec((BN, CP), lambda i: (i, 0)),
            pl.BlockSpec((NC, BN, D), lambda i: (0, i, 0)),
            pl.BlockSpec((NC, BN, D), lambda i: (0, i, 0)),
            pl.BlockSpec((NC, BN, CP), lambda i: (0, i, 0)),
            pl.BlockSpec((NC, BN, CP), lambda i: (0, i, 0)),
            pl.BlockSpec((D, H), lambda i: (0, 0)),
            pl.BlockSpec((H, H), lambda i: (0, 0)),
            pl.BlockSpec((1, H), lambda i: (0, 0)),
            pl.BlockSpec((H, D), lambda i: (0, 0)),
            pl.BlockSpec((1, D), lambda i: (0, 0)),
        ],
        out_specs=[
            pl.BlockSpec((BN, D), lambda i: (i, 0)),
            pl.BlockSpec((BN, CP), lambda i: (i, 0)),
        ],
        out_shape=[jax.ShapeDtypeStruct((N, D), _f32),
                   jax.ShapeDtypeStruct((N, CP), _f32)],
        compiler_params=pltpu.CompilerParams(
            dimension_semantics=("arbitrary",)),
    )(nodes, coordp, aggf[0], aggf[1], aggc[0], aggc[1],
      nW1a, nW1b, nb1, nW2, nb2)


def kernel(nodes, coord, edges, edge_attr,
           eW1, eb1, eW2, eb2, aW1, ab1, aW2, ab2,
           cW1, cb1, cW2, cb2, nW1, nb1, nW2, nb2):
    row = edges[0]
    col = edges[1]
    coordp = jnp.pad(coord, ((0, 0), (0, CP - 3)))
    cx = coord[:, 0]
    cy = coord[:, 1]
    cz = coord[:, 2]

    bf = jnp.bfloat16
    Wsa = jnp.concatenate([eW1[:D], aW1[:D]], axis=1).astype(bf)
    Wda = jnp.concatenate([eW1[D:2 * D], aW1[D:2 * D]], axis=1).astype(bf)
    Wea = jnp.concatenate([eW1[2 * D + 1:], aW1[2 * D:]], axis=1).astype(bf)
    b1a = jnp.concatenate([eb1, ab1])[None, :]
    rw = jnp.concatenate([eW1[2 * D:2 * D + 1],
                          jnp.zeros((1, H), _f32)], axis=1)
    w = (*_pack_consts(), Wsa, Wda, Wea, b1a, rw, eW2.astype(bf),
         eb2[None, :], aW2.astype(bf), ab2[None, :],
         cW1.astype(bf), cb1[None, :], cW2.astype(bf), cb2[None, :])

    zf = jnp.zeros((N, D), _f32)
    zc = jnp.zeros((N, CP), _f32)

    aggf, aggc = [], []
    for h in range(NH):
        eoff = h * EH
        src, dst, rad8 = _sc_gather(nodes, cx, cy, cz, row, col, eoff)
        ef, c8 = _edge_mlp(src, dst, rad8, edge_attr, w, hoff=h * NBLK)
        aggf.append(_sc_scatter_f(ef, row, zf, eoff))
        aggc.append(_sc_scatter_c(c8, row, col, cx, cy, cz, zc, eoff))

    nodes_out, coordp_out = _node_mlp(nodes, coordp, aggf, aggc,
                                      nW1[:D].astype(bf), nW1[D:].astype(bf),
                                      nb1[None, :], nW2.astype(bf),
                                      nb2[None, :])
    return (nodes_out, coordp_out[:, :3])


# speedup vs baseline: 1.1172x; 1.1172x over previous
import jax
import jax.numpy as jnp
from jax import lax
from jax.experimental import pallas as pl
from jax.experimental.pallas import tpu as pltpu
from jax.experimental.pallas import tpu_sc as plsc

N = 10000
E = 320000
D = 128
H = 128
DE = 16
CP = 16

NC, NS = 2, 16
NW = NC * NS
NH = 2
EH = E // NH
CH = 128
NCHT = EH // CH
KMAX = -(-NCHT // NW)
BE = 1280
GB = BE // CH
NBLK = EH // BE
L = 16

_f32 = jnp.float32
_i32 = jnp.int32


def _worker_id():
    return lax.axis_index("s") * NC + lax.axis_index("c")


def _mesh():
    return plsc.VectorSubcoreMesh(core_axis_name="c", subcore_axis_name="s",
                                  num_cores=NC, num_subcores=NS)


def _make_gather_body(eoff):
    def body(nodes_h, cx_h, cy_h, cz_h, row_h, col_h,
             src_o, dst_o, rad_o,
             cx, cy, cz, ir, ic, b1, b2, rbuf, s1, s2, s3, s4, s5):
        wid = _worker_id()
        s = lax.axis_index("s")
        pltpu.sync_copy(cx_h, cx)
        pltpu.sync_copy(cy_h, cy)
        pltpu.sync_copy(cz_h, cz)

        def loop(k, _):
            cid = k * NW + wid

            @pl.when(cid < NCHT)
            def _():
                base = cid * CH
                di1 = pltpu.async_copy(row_h.at[pl.ds(eoff + base, CH)], ir, s3)
                di2 = pltpu.async_copy(col_h.at[pl.ds(eoff + base, CH)], ic, s4)
                di1.wait()
                di2.wait()
                d1 = pltpu.async_copy(nodes_h.at[ir], b1, s1)
                d2 = pltpu.async_copy(nodes_h.at[ic], b2, s2)
                for g in range(CH // L):
                    ivr = ir[pl.ds(g * L, L)]
                    ivc = ic[pl.ds(g * L, L)]
                    dx = plsc.load_gather(cx, [ivr]) - plsc.load_gather(cx, [ivc])
                    dy = plsc.load_gather(cy, [ivr]) - plsc.load_gather(cy, [ivc])
                    dz = plsc.load_gather(cz, [ivr]) - plsc.load_gather(cz, [ivc])
                    rbuf[0, pl.ds(g * L, L)] = dx * dx + dy * dy + dz * dz
                dr = pltpu.async_copy(
                    rbuf, rad_o.at[cid // GB].at[pl.ds(cid % GB, 1)], s5)
                d1.wait()
                w1 = pltpu.async_copy(b1, src_o.at[pl.ds(base, CH)], s3)
                d2.wait()
                w2 = pltpu.async_copy(b2, dst_o.at[pl.ds(base, CH)], s4)
                dr.wait()
                w1.wait()
                w2.wait()
            return _

        lax.fori_loop(0, KMAX, loop, None)

    return body


def _sc_gather(nodes, cx, cy, cz, row, col, eoff):
    fn = pl.kernel(
        _make_gather_body(eoff),
        out_type=[
            jax.ShapeDtypeStruct((EH, D), _f32),
            jax.ShapeDtypeStruct((EH, D), _f32),
            jax.ShapeDtypeStruct((NBLK, GB, CH), _f32),
        ],
        mesh=_mesh(),
        scratch_types=[
            pltpu.VMEM((N,), _f32), pltpu.VMEM((N,), _f32),
            pltpu.VMEM((N,), _f32),
            pltpu.VMEM((CH,), _i32), pltpu.VMEM((CH,), _i32),
            pltpu.VMEM((CH, D), _f32), pltpu.VMEM((CH, D), _f32),
            pltpu.VMEM((1, CH), _f32),
            pltpu.SemaphoreType.DMA, pltpu.SemaphoreType.DMA,
            pltpu.SemaphoreType.DMA, pltpu.SemaphoreType.DMA,
            pltpu.SemaphoreType.DMA,
        ],
        compiler_params=pltpu.CompilerParams(needs_layout_passes=False),
    )
    return fn(nodes, cx, cy, cz, row, col)


def _edge_mlp_body(src_r, dst_r, rad_r, ea_r,
                   eW1s_r, eW1d_r, eW1r_r, eW1e_r, eb1_r, eW2_r, eb2_r,
                   aW1s_r, aW1d_r, aW1e_r, ab1_r, aW2_r, ab2_r,
                   cW1_r, cb1_r, cW2_r, cb2_r,
                   ef_o, c8_o):
    src = src_r[...].astype(jnp.bfloat16)
    dst = dst_r[...].astype(jnp.bfloat16)
    ea = ea_r[...].astype(jnp.bfloat16)

    rads = []
    for g in range(GB):
        rowv = rad_r[0, g:g + 1, :]
        rads.append(jnp.broadcast_to(rowv, (CH, CH)).T)
    radf = jnp.concatenate(rads, axis=0)

    def mm(a, b):
        return jnp.dot(a.astype(jnp.bfloat16), b, preferred_element_type=_f32)

    def tshrink(x):
        return x - jnp.tanh(x)

    h = (mm(src, eW1s_r[...]) + mm(dst, eW1d_r[...]) + radf * eW1r_r[...]
         + mm(ea, eW1e_r[...]) + eb1_r[...])
    h = tshrink(h)
    h = tshrink(mm(h, eW2_r[...]) + eb2_r[...])

    a = (mm(src, aW1s_r[...]) + mm(dst, aW1d_r[...]) + mm(ea, aW1e_r[...])
         + ab1_r[...])
    a = tshrink(a)
    a = mm(a, aW2_r[...]) + ab2_r[...]
    a = 1.0 / (1.0 + jnp.exp(-a))

    ef = h * a
    cc = tshrink(mm(ef, cW1_r[...]) + cb1_r[...])
    cc = mm(cc, cW2_r[...]) + cb2_r[...]

    ef_o[...] = ef
    rows = []
    for g in range(GB):
        cg = cc[g * CH:(g + 1) * CH, :]
        rows.append(jnp.broadcast_to(cg, (CH, CH)).T[0:1, :])
    c8_o[0] = jnp.concatenate(rows, axis=0)


def _edge_mlp(src, dst, rad8, edge_attr, w, hoff):
    grid = (NBLK,)

    def eb(i):
        return (i, 0)

    def wspec(shape):
        return pl.BlockSpec(shape, lambda i: (0, 0))

    in_specs = [pl.BlockSpec((BE, D), eb), pl.BlockSpec((BE, D), eb),
                pl.BlockSpec((1, GB, CH), lambda i: (i, 0, 0)),
                pl.BlockSpec((BE, DE), lambda i: (i + hoff, 0)),
                wspec((D, H)), wspec((D, H)), wspec((1, H)), wspec((DE, H)),
                wspec((1, H)), wspec((H, H)), wspec((1, H)),
                wspec((D, H)), wspec((D, H)), wspec((DE, H)), wspec((1, H)),
                wspec((H, 1)), wspec((1, 1)),
                wspec((H, H)), wspec((1, H)), wspec((H, 1)), wspec((1, 1))]
    out_specs = [pl.BlockSpec((BE, D), eb),
                 pl.BlockSpec((1, GB, CH), lambda i: (i, 0, 0))]
    return pl.pallas_call(
        _edge_mlp_body,
        grid=grid,
        in_specs=in_specs,
        out_specs=out_specs,
        out_shape=[jax.ShapeDtypeStruct((EH, D), _f32),
                   jax.ShapeDtypeStruct((NBLK, GB, CH), _f32)],
        compiler_params=pltpu.CompilerParams(
            dimension_semantics=("arbitrary",)),
    )(src, dst, rad8, edge_attr, *w)


RP = 624
RREM = N - NS * RP


def _acc_init(z_h, acc, s):
    pltpu.sync_copy(z_h.at[pl.ds(s * RP, RP)], acc.at[pl.ds(s * RP, RP)])

    @pl.when(s == NS - 1)
    def _():
        pltpu.sync_copy(z_h.at[pl.ds(NS * RP, RREM)],
                        acc.at[pl.ds(NS * RP, RREM)])


def _acc_writeout(acc, out_o, c, s):
    pltpu.sync_copy(acc.at[pl.ds(s * RP, RP)],
                    out_o.at[c].at[pl.ds(s * RP, RP)])

    @pl.when(s == NS - 1)
    def _():
        pltpu.sync_copy(acc.at[pl.ds(NS * RP, RREM)],
                        out_o.at[c].at[pl.ds(NS * RP, RREM)])


def _make_scatter_f_body(eoff):
    def body(v_h, row_h, z_h, out_o, acc, idx, buf, s1, s2):
        c = lax.axis_index("c")
        s = lax.axis_index("s")
        wid = _worker_id()

        _acc_init(z_h, acc, s)
        plsc.subcore_barrier()

        def loop(k, _):
            cid = k * NW + wid

            @pl.when(cid < NCHT)
            def _():
                base = cid * CH
                di = pltpu.async_copy(row_h.at[pl.ds(eoff + base, CH)], idx, s2)
                dv = pltpu.async_copy(v_h.at[pl.ds(base, CH)], buf, s1)
                di.wait()
                dv.wait()
                pltpu.sync_copy(buf, acc.at[idx], add=True)
            return _

        lax.fori_loop(0, KMAX, loop, None)
        plsc.subcore_barrier()
        _acc_writeout(acc, out_o, c, s)

    return body


def _sc_scatter_f(ef, row, zf, eoff):
    fn = pl.kernel(
        _make_scatter_f_body(eoff),
        out_type=[jax.ShapeDtypeStruct((NC, N, D), _f32)],
        mesh=_mesh(),
        scratch_types=[
            pltpu.VMEM_SHARED((N, D), _f32),
            pltpu.VMEM((CH,), _i32),
            pltpu.VMEM((CH, D), _f32),
            pltpu.SemaphoreType.DMA, pltpu.SemaphoreType.DMA,
        ],
    )
    return fn(ef, row, zf)[0]


def _make_scatter_c_body(eoff):
    def body(c8_h, row_h, col_h, cx_h, cy_h, cz_h, z_h, out_o,
             acc, cx, cy, cz, ir, ic, cbuf, trbuf, s1, s2, s3):
        c = lax.axis_index("c")
        s = lax.axis_index("s")
        wid = _worker_id()

        pltpu.sync_copy(cx_h, cx)
        pltpu.sync_copy(cy_h, cy)
        pltpu.sync_copy(cz_h, cz)
        _acc_init(z_h, acc, s)
        plsc.subcore_barrier()

        iota = lax.iota(_i32, L)

        def loop(k, _):
            cid = k * NW + wid

            @pl.when(cid < NCHT)
            def _():
                base = cid * CH
                d1 = pltpu.async_copy(row_h.at[pl.ds(eoff + base, CH)], ir, s1)
                d2 = pltpu.async_copy(col_h.at[pl.ds(eoff + base, CH)], ic, s2)
                d3 = pltpu.async_copy(
                    c8_h.at[cid // GB].at[pl.ds(cid % GB, 1)], cbuf, s3)
                d1.wait()
                d2.wait()
                d3.wait()
                for g in range(CH // L):
                    ivr = ir[pl.ds(g * L, L)]
                    ivc = ic[pl.ds(g * L, L)]
                    cv = cbuf[0, pl.ds(g * L, L)]
                    tx = (plsc.load_gather(cx, [ivr])
                          - plsc.load_gather(cx, [ivc])) * cv
                    ty = (plsc.load_gather(cy, [ivr])
                          - plsc.load_gather(cy, [ivc])) * cv
                    tz = (plsc.load_gather(cz, [ivr])
                          - plsc.load_gather(cz, [ivc])) * cv
                    ridx = iota + (g * L)
                    plsc.store_scatter(trbuf, [ridx, jnp.full((L,), 0, _i32)], tx)
                    plsc.store_scatter(trbuf, [ridx, jnp.full((L,), 1, _i32)], ty)
                    plsc.store_scatter(trbuf, [ridx, jnp.full((L,), 2, _i32)], tz)
                pltpu.sync_copy(trbuf, acc.at[ir], add=True)
            return _

        lax.fori_loop(0, KMAX, loop, None)
        plsc.subcore_barrier()
        _acc_writeout(acc, out_o, c, s)

    return body


def _sc_scatter_c(c8, row, col, cx, cy, cz, zc, eoff):
    fn = pl.kernel(
        _make_scatter_c_body(eoff),
        out_type=[jax.ShapeDtypeStruct((NC, N, CP), _f32)],
        mesh=_mesh(),
        scratch_types=[
            pltpu.VMEM_SHARED((N, CP), _f32),
            pltpu.VMEM((N,), _f32), pltpu.VMEM((N,), _f32),
            pltpu.VMEM((N,), _f32),
            pltpu.VMEM((CH,), _i32), pltpu.VMEM((CH,), _i32),
            pltpu.VMEM((1, CH), _f32),
            pltpu.VMEM((CH, CP), _f32),
            pltpu.SemaphoreType.DMA, pltpu.SemaphoreType.DMA,
            pltpu.SemaphoreType.DMA,
        ],
        compiler_params=pltpu.CompilerParams(use_tc_tiling_on_sc=False,
                                             needs_layout_passes=False),
    )
    return fn(c8, row, col, cx, cy, cz, zc)[0]


def _node_body(nodes_r, coordp_r, f0_r, f1_r, c0_r, c1_r,
               nW1a_r, nW1b_r, nb1_r, nW2_r, nb2_r,
               nodes_o, coordp_o):
    nodes = nodes_r[...]
    aggf = f0_r[0] + f0_r[1] + f1_r[0] + f1_r[1]

    def mm(a, b):
        return jnp.dot(a.astype(jnp.bfloat16), b, preferred_element_type=_f32)

    n = mm(nodes, nW1a_r[...]) + mm(aggf, nW1b_r[...]) + nb1_r[...]
    n = n - jnp.tanh(n)
    n = mm(n, nW2_r[...]) + nb2_r[...]
    nodes_o[...] = nodes + n
    coordp_o[...] = coordp_r[...] + c0_r[0] + c0_r[1] + c1_r[0] + c1_r[1]


def _node_mlp(nodes, coordp, aggf, aggc, nW1a, nW1b, nb1, nW2, nb2):
    BN = 2000
    grid = (N // BN,)
    return pl.pallas_call(
        _node_body,
        grid=grid,
        in_specs=[
            pl.BlockSpec((BN, D), lambda i: (i, 0)),
            pl.BlockSpec((BN, CP), lambda i: (i, 0)),
            pl.BlockSpec((NC, BN, D), lambda i: (0, i, 0)),
            pl.BlockSpec((NC, BN, D), lambda i: (0, i, 0)),
            pl.BlockSpec((NC, BN, CP), lambda i: (0, i, 0)),
            pl.BlockSpec((NC, BN, CP), lambda i: (0, i, 0)),
            pl.BlockSpec((D, H), lambda i: (0, 0)),
            pl.BlockSpec((H, H), lambda i: (0, 0)),
            pl.BlockSpec((1, H), lambda i: (0, 0)),
            pl.BlockSpec((H, D), lambda i: (0, 0)),
            pl.BlockSpec((1, D), lambda i: (0, 0)),
        ],
        out_specs=[
            pl.BlockSpec((BN, D), lambda i: (i, 0)),
            pl.BlockSpec((BN, CP), lambda i: (i, 0)),
        ],
        out_shape=[jax.ShapeDtypeStruct((N, D), _f32),
                   jax.ShapeDtypeStruct((N, CP), _f32)],
        compiler_params=pltpu.CompilerParams(
            dimension_semantics=("arbitrary",)),
    )(nodes, coordp, aggf[0], aggf[1], aggc[0], aggc[1],
      nW1a, nW1b, nb1, nW2, nb2)


def kernel(nodes, coord, edges, edge_attr,
           eW1, eb1, eW2, eb2, aW1, ab1, aW2, ab2,
           cW1, cb1, cW2, cb2, nW1, nb1, nW2, nb2):
    row = edges[0]
    col = edges[1]
    coordp = jnp.pad(coord, ((0, 0), (0, CP - 3)))
    cx = coord[:, 0]
    cy = coord[:, 1]
    cz = coord[:, 2]

    bf = jnp.bfloat16
    w = (eW1[:D].astype(bf), eW1[D:2 * D].astype(bf),
         eW1[2 * D:2 * D + 1], eW1[2 * D + 1:].astype(bf),
         eb1[None, :], eW2.astype(bf), eb2[None, :],
         aW1[:D].astype(bf), aW1[D:2 * D].astype(bf), aW1[2 * D:].astype(bf),
         ab1[None, :], aW2.astype(bf), ab2[None, :],
         cW1.astype(bf), cb1[None, :], cW2.astype(bf), cb2[None, :])

    zf = jnp.zeros((N, D), _f32)
    zc = jnp.zeros((N, CP), _f32)

    aggf, aggc = [], []
    for h in range(NH):
        eoff = h * EH
        src, dst, rad8 = _sc_gather(nodes, cx, cy, cz, row, col, eoff)
        ef, c8 = _edge_mlp(src, dst, rad8, edge_attr, w, hoff=h * NBLK)
        aggf.append(_sc_scatter_f(ef, row, zf, eoff))
        aggc.append(_sc_scatter_c(c8, row, col, cx, cy, cz, zc, eoff))

    nodes_out, coordp_out = _node_mlp(nodes, coordp, aggf, aggc,
                                      nW1[:D].astype(bf), nW1[D:].astype(bf),
                                      nb1[None, :], nW2.astype(bf),
                                      nb2[None, :])
    return (nodes_out, coordp_out[:, :3])


# speedup vs baseline: 1.1441x; 1.0241x over previous
import jax
import jax.numpy as jnp
from jax import lax
from jax.experimental import pallas as pl
from jax.experimental.pallas import tpu as pltpu
from jax.experimental.pallas import tpu_sc as plsc

N = 10000
E = 320000
D = 128
H = 128
DE = 16
CP = 16

NC, NS = 2, 16
NW = NC * NS
NH = 2
EH = E // NH
CH = 128
NCHT = EH // CH
KMAX = -(-NCHT // NW)
BE = 1280
GB = BE // CH
NBLK = EH // BE
L = 16

_f32 = jnp.float32
_i32 = jnp.int32


def _worker_id():
    return lax.axis_index("s") * NC + lax.axis_index("c")


def _mesh():
    return plsc.VectorSubcoreMesh(core_axis_name="c", subcore_axis_name="s",
                                  num_cores=NC, num_subcores=NS)


def _make_gather_body(eoff):
    def body(nodes_h, cx_h, cy_h, cz_h, row_h, col_h,
             src_o, dst_o, rad_o,
             cx, cy, cz, ir, ic, b1, b2, rbuf, s1, s2, s3, s4, s5):
        wid = _worker_id()
        s = lax.axis_index("s")
        pltpu.sync_copy(cx_h, cx)
        pltpu.sync_copy(cy_h, cy)
        pltpu.sync_copy(cz_h, cz)

        def loop(k, _):
            cid = k * NW + wid

            @pl.when(cid < NCHT)
            def _():
                base = cid * CH
                di1 = pltpu.async_copy(row_h.at[pl.ds(eoff + base, CH)], ir, s3)
                di2 = pltpu.async_copy(col_h.at[pl.ds(eoff + base, CH)], ic, s4)
                di1.wait()
                di2.wait()
                d1 = pltpu.async_copy(nodes_h.at[ir], b1, s1)
                d2 = pltpu.async_copy(nodes_h.at[ic], b2, s2)
                for g in range(CH // L):
                    ivr = ir[pl.ds(g * L, L)]
                    ivc = ic[pl.ds(g * L, L)]
                    dx = plsc.load_gather(cx, [ivr]) - plsc.load_gather(cx, [ivc])
                    dy = plsc.load_gather(cy, [ivr]) - plsc.load_gather(cy, [ivc])
                    dz = plsc.load_gather(cz, [ivr]) - plsc.load_gather(cz, [ivc])
                    rbuf[0, pl.ds(g * L, L)] = dx * dx + dy * dy + dz * dz
                dr = pltpu.async_copy(
                    rbuf, rad_o.at[cid // GB].at[pl.ds(cid % GB, 1)], s5)
                d1.wait()
                w1 = pltpu.async_copy(b1, src_o.at[pl.ds(base, CH)], s3)
                d2.wait()
                w2 = pltpu.async_copy(b2, dst_o.at[pl.ds(base, CH)], s4)
                dr.wait()
                w1.wait()
                w2.wait()
            return _

        lax.fori_loop(0, KMAX, loop, None)

    return body


def _sc_gather(nodes, cx, cy, cz, row, col, eoff):
    fn = pl.kernel(
        _make_gather_body(eoff),
        out_type=[
            jax.ShapeDtypeStruct((EH, D), _f32),
            jax.ShapeDtypeStruct((EH, D), _f32),
            jax.ShapeDtypeStruct((NBLK, GB, CH), _f32),
        ],
        mesh=_mesh(),
        scratch_types=[
            pltpu.VMEM((N,), _f32), pltpu.VMEM((N,), _f32),
            pltpu.VMEM((N,), _f32),
            pltpu.VMEM((CH,), _i32), pltpu.VMEM((CH,), _i32),
            pltpu.VMEM((CH, D), _f32), pltpu.VMEM((CH, D), _f32),
            pltpu.VMEM((1, CH), _f32),
            pltpu.SemaphoreType.DMA, pltpu.SemaphoreType.DMA,
            pltpu.SemaphoreType.DMA, pltpu.SemaphoreType.DMA,
            pltpu.SemaphoreType.DMA,
        ],
        compiler_params=pltpu.CompilerParams(needs_layout_passes=False),
    )
    return fn(nodes, cx, cy, cz, row, col)


def _edge_mlp_body(src_r, dst_r, rad_r, ea_r,
                   eW1s_r, eW1d_r, eW1r_r, eW1e_r, eb1_r, eW2_r, eb2_r,
                   aW1s_r, aW1d_r, aW1e_r, ab1_r, aW2_r, ab2_r,
                   cW1_r, cb1_r, cW2_r, cb2_r,
                   ef_o, c8_o):
    def mm(a, b):
        return jnp.dot(a.astype(jnp.bfloat16), b, preferred_element_type=_f32)

    def tshrink(x):
        return x - jnp.tanh(x)

    SUB = 1
    GS = GB // SUB
    BS = GS * CH
    for sub in range(SUB):
        rs = pl.ds(sub * BS, BS)
        src = src_r[rs, :].astype(jnp.bfloat16)
        dst = dst_r[rs, :].astype(jnp.bfloat16)
        ea = ea_r[rs, :].astype(jnp.bfloat16)

        rads = []
        for g in range(sub * GS, (sub + 1) * GS):
            rowv = rad_r[0, g:g + 1, :]
            rads.append(jnp.broadcast_to(rowv, (CH, CH)).T)
        radf = jnp.concatenate(rads, axis=0)

        h = (mm(src, eW1s_r[...]) + mm(dst, eW1d_r[...]) + radf * eW1r_r[...]
             + mm(ea, eW1e_r[...]) + eb1_r[...])
        h = tshrink(h)
        h = tshrink(mm(h, eW2_r[...]) + eb2_r[...])

        a = (mm(src, aW1s_r[...]) + mm(dst, aW1d_r[...]) + mm(ea, aW1e_r[...])
             + ab1_r[...])
        a = tshrink(a)
        a = mm(a, aW2_r[...]) + ab2_r[...]
        a = 0.5 + 0.5 * jnp.tanh(0.5 * a)

        ef = h * a
        cc = tshrink(mm(ef, cW1_r[...]) + cb1_r[...])
        cc = mm(cc, cW2_r[...]) + cb2_r[...]

        ef_o[rs, :] = ef
        rows = []
        for g in range(GS):
            cg = cc[g * CH:(g + 1) * CH, :]
            rows.append(jnp.broadcast_to(cg, (CH, CH)).T[0:1, :])
        c8_o[0, pl.ds(sub * GS, GS), :] = jnp.concatenate(rows, axis=0)


def _edge_mlp(src, dst, rad8, edge_attr, w, hoff):
    grid = (NBLK,)

    def eb(i):
        return (i, 0)

    def wspec(shape):
        return pl.BlockSpec(shape, lambda i: (0, 0))

    in_specs = [pl.BlockSpec((BE, D), eb), pl.BlockSpec((BE, D), eb),
                pl.BlockSpec((1, GB, CH), lambda i: (i, 0, 0)),
                pl.BlockSpec((BE, DE), lambda i: (i + hoff, 0)),
                wspec((D, H)), wspec((D, H)), wspec((1, H)), wspec((DE, H)),
                wspec((1, H)), wspec((H, H)), wspec((1, H)),
                wspec((D, H)), wspec((D, H)), wspec((DE, H)), wspec((1, H)),
                wspec((H, 1)), wspec((1, 1)),
                wspec((H, H)), wspec((1, H)), wspec((H, 1)), wspec((1, 1))]
    out_specs = [pl.BlockSpec((BE, D), eb),
                 pl.BlockSpec((1, GB, CH), lambda i: (i, 0, 0))]
    return pl.pallas_call(
        _edge_mlp_body,
        grid=grid,
        in_specs=in_specs,
        out_specs=out_specs,
        out_shape=[jax.ShapeDtypeStruct((EH, D), _f32),
                   jax.ShapeDtypeStruct((NBLK, GB, CH), _f32)],
        compiler_params=pltpu.CompilerParams(
            dimension_semantics=("arbitrary",)),
    )(src, dst, rad8, edge_attr, *w)


RP = 624
RREM = N - NS * RP


def _acc_init(z_h, acc, s):
    pltpu.sync_copy(z_h.at[pl.ds(s * RP, RP)], acc.at[pl.ds(s * RP, RP)])

    @pl.when(s == NS - 1)
    def _():
        pltpu.sync_copy(z_h.at[pl.ds(NS * RP, RREM)],
                        acc.at[pl.ds(NS * RP, RREM)])


def _acc_writeout(acc, out_o, c, s):
    pltpu.sync_copy(acc.at[pl.ds(s * RP, RP)],
                    out_o.at[c].at[pl.ds(s * RP, RP)])

    @pl.when(s == NS - 1)
    def _():
        pltpu.sync_copy(acc.at[pl.ds(NS * RP, RREM)],
                        out_o.at[c].at[pl.ds(NS * RP, RREM)])


KPAIR = (KMAX - 2) // 2


def _make_scatter_f_body(eoff):
    def body(v_h, row_h, z_h, out_o, acc, idx, buf, idx2, buf2, s1, s2, s3, s4):
        c = lax.axis_index("c")
        s = lax.axis_index("s")
        wid = _worker_id()

        _acc_init(z_h, acc, s)
        plsc.subcore_barrier()

        def start(cid, ib, vb, si, sv):
            base = cid * CH
            di = pltpu.async_copy(row_h.at[pl.ds(eoff + base, CH)], ib, si)
            dv = pltpu.async_copy(v_h.at[pl.ds(base, CH)], vb, sv)
            return di, dv

        def loop(j, _):
            cida = (2 * j) * NW + wid
            cidb = (2 * j + 1) * NW + wid
            da = start(cida, idx, buf, s2, s1)
            db = start(cidb, idx2, buf2, s4, s3)
            da[0].wait()
            da[1].wait()
            pltpu.sync_copy(buf, acc.at[idx], add=True)
            db[0].wait()
            db[1].wait()
            pltpu.sync_copy(buf2, acc.at[idx2], add=True)
            return _

        lax.fori_loop(0, KPAIR, loop, None)

        for k in range(2 * KPAIR, KMAX):
            cid = k * NW + wid

            @pl.when(cid < NCHT)
            def _():
                d = start(cid, idx, buf, s2, s1)
                d[0].wait()
                d[1].wait()
                pltpu.sync_copy(buf, acc.at[idx], add=True)

        plsc.subcore_barrier()
        _acc_writeout(acc, out_o, c, s)

    return body


def _sc_scatter_f(ef, row, zf, eoff):
    fn = pl.kernel(
        _make_scatter_f_body(eoff),
        out_type=[jax.ShapeDtypeStruct((NC, N, D), _f32)],
        mesh=_mesh(),
        scratch_types=[
            pltpu.VMEM_SHARED((N, D), _f32),
            pltpu.VMEM((CH,), _i32),
            pltpu.VMEM((CH, D), _f32),
            pltpu.VMEM((CH,), _i32),
            pltpu.VMEM((CH, D), _f32),
            pltpu.SemaphoreType.DMA, pltpu.SemaphoreType.DMA,
            pltpu.SemaphoreType.DMA, pltpu.SemaphoreType.DMA,
        ],
    )
    return fn(ef, row, zf)[0]


def _make_scatter_c_body(eoff):
    def body(c8_h, row_h, col_h, cx_h, cy_h, cz_h, z_h, out_o,
             acc, cx, cy, cz, ir, ic, cbuf, trbuf, s1, s2, s3):
        c = lax.axis_index("c")
        s = lax.axis_index("s")
        wid = _worker_id()

        pltpu.sync_copy(cx_h, cx)
        pltpu.sync_copy(cy_h, cy)
        pltpu.sync_copy(cz_h, cz)
        _acc_init(z_h, acc, s)
        plsc.subcore_barrier()

        iota = lax.iota(_i32, L)

        def loop(k, _):
            cid = k * NW + wid

            @pl.when(cid < NCHT)
            def _():
                base = cid * CH
                d1 = pltpu.async_copy(row_h.at[pl.ds(eoff + base, CH)], ir, s1)
                d2 = pltpu.async_copy(col_h.at[pl.ds(eoff + base, CH)], ic, s2)
                d3 = pltpu.async_copy(
                    c8_h.at[cid // GB].at[pl.ds(cid % GB, 1)], cbuf, s3)
                d1.wait()
                d2.wait()
                d3.wait()
                for g in range(CH // L):
                    ivr = ir[pl.ds(g * L, L)]
                    ivc = ic[pl.ds(g * L, L)]
                    cv = cbuf[0, pl.ds(g * L, L)]
                    tx = (plsc.load_gather(cx, [ivr])
                          - plsc.load_gather(cx, [ivc])) * cv
                    ty = (plsc.load_gather(cy, [ivr])
                          - plsc.load_gather(cy, [ivc])) * cv
                    tz = (plsc.load_gather(cz, [ivr])
                          - plsc.load_gather(cz, [ivc])) * cv
                    ridx = iota + (g * L)
                    plsc.store_scatter(trbuf, [ridx, jnp.full((L,), 0, _i32)], tx)
                    plsc.store_scatter(trbuf, [ridx, jnp.full((L,), 1, _i32)], ty)
                    plsc.store_scatter(trbuf, [ridx, jnp.full((L,), 2, _i32)], tz)
                pltpu.sync_copy(trbuf, acc.at[ir], add=True)
            return _

        lax.fori_loop(0, KMAX, loop, None)
        plsc.subcore_barrier()
        _acc_writeout(acc, out_o, c, s)

    return body


def _sc_scatter_c(c8, row, col, cx, cy, cz, zc, eoff):
    fn = pl.kernel(
        _make_scatter_c_body(eoff),
        out_type=[jax.ShapeDtypeStruct((NC, N, CP), _f32)],
        mesh=_mesh(),
        scratch_types=[
            pltpu.VMEM_SHARED((N, CP), _f32),
            pltpu.VMEM((N,), _f32), pltpu.VMEM((N,), _f32),
            pltpu.VMEM((N,), _f32),
            pltpu.VMEM((CH,), _i32), pltpu.VMEM((CH,), _i32),
            pltpu.VMEM((1, CH), _f32),
            pltpu.VMEM((CH, CP), _f32),
            pltpu.SemaphoreType.DMA, pltpu.SemaphoreType.DMA,
            pltpu.SemaphoreType.DMA,
        ],
        compiler_params=pltpu.CompilerParams(use_tc_tiling_on_sc=False,
                                             needs_layout_passes=False),
    )
    return fn(c8, row, col, cx, cy, cz, zc)[0]


def _node_body(nodes_r, coordp_r, f0_r, f1_r, c0_r, c1_r,
               nW1a_r, nW1b_r, nb1_r, nW2_r, nb2_r,
               nodes_o, coordp_o):
    nodes = nodes_r[...]
    aggf = f0_r[0] + f0_r[1] + f1_r[0] + f1_r[1]

    def mm(a, b):
        return jnp.dot(a.astype(jnp.bfloat16), b, preferred_element_type=_f32)

    n = mm(nodes, nW1a_r[...]) + mm(aggf, nW1b_r[...]) + nb1_r[...]
    n = n - jnp.tanh(n)
    n = mm(n, nW2_r[...]) + nb2_r[...]
    nodes_o[...] = nodes + n
    coordp_o[...] = coordp_r[...] + c0_r[0] + c0_r[1] + c1_r[0] + c1_r[1]


def _node_mlp(nodes, coordp, aggf, aggc, nW1a, nW1b, nb1, nW2, nb2):
    BN = 2000
    grid = (N // BN,)
    return pl.pallas_call(
        _node_body,
        grid=grid,
        in_specs=[
            pl.BlockSpec((BN, D), lambda i: (i, 0)),
            pl.BlockSpec((BN, CP), lambda i: (i, 0)),
            pl.BlockSpec((NC, BN, D), lambda i: (0, i, 0)),
            pl.BlockSpec((NC, BN, D), lambda i: (0, i, 0)),
            pl.BlockSpec((NC, BN, CP), lambda i: (0, i, 0)),
            pl.BlockSpec((NC, BN, CP), lambda i: (0, i, 0)),
            pl.BlockSpec((D, H), lambda i: (0, 0)),
            pl.BlockSpec((H, H), lambda i: (0, 0)),
            pl.BlockSpec((1, H), lambda i: (0, 0)),
            pl.BlockSpec((H, D), lambda i: (0, 0)),
            pl.BlockSpec((1, D), lambda i: (0, 0)),
        ],
        out_specs=[
            pl.BlockSpec((BN, D), lambda i: (i, 0)),
            pl.BlockSpec((BN, CP), lambda i: (i, 0)),
        ],
        out_shape=[jax.ShapeDtypeStruct((N, D), _f32),
                   jax.ShapeDtypeStruct((N, CP), _f32)],
        compiler_params=pltpu.CompilerParams(
            dimension_semantics=("arbitrary",)),
    )(nodes, coordp, aggf[0], aggf[1], aggc[0], aggc[1],
      nW1a, nW1b, nb1, nW2, nb2)


def kernel(nodes, coord, edges, edge_attr,
           eW1, eb1, eW2, eb2, aW1, ab1, aW2, ab2,
           cW1, cb1, cW2, cb2, nW1, nb1, nW2, nb2):
    row = edges[0]
    col = edges[1]
    coordp = jnp.pad(coord, ((0, 0), (0, CP - 3)))
    cx = coord[:, 0]
    cy = coord[:, 1]
    cz = coord[:, 2]

    bf = jnp.bfloat16
    w = (eW1[:D].astype(bf), eW1[D:2 * D].astype(bf),
         eW1[2 * D:2 * D + 1], eW1[2 * D + 1:].astype(bf),
         eb1[None, :], eW2.astype(bf), eb2[None, :],
         aW1[:D].astype(bf), aW1[D:2 * D].astype(bf), aW1[2 * D:].astype(bf),
         ab1[None, :], aW2.astype(bf), ab2[None, :],
         cW1.astype(bf), cb1[None, :], cW2.astype(bf), cb2[None, :])

    zf = jnp.zeros((N, D), _f32)
    zc = jnp.zeros((N, CP), _f32)

    aggf, aggc = [], []
    for h in range(NH):
        eoff = h * EH
        src, dst, rad8 = _sc_gather(nodes, cx, cy, cz, row, col, eoff)
        ef, c8 = _edge_mlp(src, dst, rad8, edge_attr, w, hoff=h * NBLK)
        aggf.append(_sc_scatter_f(ef, row, zf, eoff))
        aggc.append(_sc_scatter_c(c8, row, col, cx, cy, cz, zc, eoff))

    nodes_out, coordp_out = _node_mlp(nodes, coordp, aggf, aggc,
                                      nW1[:D].astype(bf), nW1[D:].astype(bf),
                                      nb1[None, :], nW2.astype(bf),
                                      nb2[None, :])
    return (nodes_out, coordp_out[:, :3])
